# baseline (speedup 1.0000x reference)
import jax
import jax.numpy as jnp
from jax import lax
from jax.experimental import pallas as pl
from jax.experimental.pallas import tpu as pltpu
from jax.experimental.pallas import tpu_sc as plsc

N = 10000
E = 320000
D_IN = 128
D_HID = 128
D_OUT = 64
DP = 144

NC = 2
NS = 16
NW = NC * NS
EPW = E // NW
CHUNK = 48
NCHUNK = 209
EPW_PAD = NCHUNK * CHUNK
N_ACC = 10016
ROWS_PER_TILE = N_ACC // NS
DCHUNKS = 64
NRING = 4


def _agg_body(h_hbm, src_hbm, dst_hbm, z_hbm, out_hbm,
              sidx_v, didx_v, r0, r1, r2, r3, acc_sh,
              g0, g1, g2, g3, s0, s1, s2, s3):
    rows = (r0, r1, r2, r3)
    gsem = (g0, g1, g2, g3)
    ssem = (s0, s1, s2, s3)
    cid = lax.axis_index("c")
    sid = lax.axis_index("s")
    wid = sid * NC + cid

    pltpu.sync_copy(z_hbm, acc_sh.at[pl.ds(sid * ROWS_PER_TILE, ROWS_PER_TILE)])

    pltpu.sync_copy(src_hbm.at[wid], sidx_v)
    pltpu.sync_copy(dst_hbm.at[wid, pl.ds(0, DCHUNKS)], didx_v)
    plsc.subcore_barrier()

    def start_gather(c, r):
        pltpu.async_copy(h_hbm.at[sidx_v.at[c]], rows[r], gsem[r])

    def wait_gather(c, r):
        pltpu.make_async_copy(h_hbm.at[sidx_v.at[c]], rows[r], gsem[r]).wait()

    def start_scatter(d, r):
        pltpu.async_copy(rows[r], acc_sh.at[didx_v.at[d]], ssem[r], add=True)

    def wait_scatter(r):
        pltpu.make_async_copy(rows[r], acc_sh.at[didx_v.at[0]], ssem[r]).wait()

    def step(c, j, dbase, do_swait=True, do_gissue=True):
        wait_gather(c, j)
        start_scatter(c - dbase, j)
        if do_swait:
            wait_scatter((j + 2) % NRING)
        if do_gissue:
            start_gather(c + 2, (j + 2) % NRING)

    start_gather(0, 0)
    start_gather(1, 1)

    stages = [(0, DCHUNKS), (DCHUNKS, DCHUNKS), (2 * DCHUNKS, DCHUNKS),
              (3 * DCHUNKS, NCHUNK - 3 * DCHUNKS)]
    for s, (base, length) in enumerate(stages):
        if s > 0:
            wait_scatter((base - 2) % NRING)
            wait_scatter((base - 1) % NRING)
            pltpu.sync_copy(dst_hbm.at[wid, pl.ds(base, length)],
                            didx_v.at[pl.ds(0, length)])
        step(base + 0, 0, base, do_swait=False)
        step(base + 1, 1, base, do_swait=False)
        step(base + 2, 2, base)
        step(base + 3, 3, base)
        nfull = length // NRING
        is_last = s == len(stages) - 1
        loop_end = nfull - 1 if is_last else nfull

        def body(i, carry, base=base):
            c = base + NRING * i
            step(c + 0, 0, base)
            step(c + 1, 1, base)
            step(c + 2, 2, base)
            step(c + 3, 3, base)
            return carry

        lax.fori_loop(1, loop_end, body, 0)
        for t in range(NRING * loop_end, length):
            c = base + t
            step(c, t % NRING, base, do_gissue=(c + 2 < NCHUNK))
    wait_scatter((NCHUNK - 2) % NRING)
    wait_scatter((NCHUNK - 1) % NRING)

    plsc.subcore_barrier()
    pltpu.sync_copy(acc_sh.at[pl.ds(sid * ROWS_PER_TILE, ROWS_PER_TILE)],
                    out_hbm.at[cid, pl.ds(sid * ROWS_PER_TILE, ROWS_PER_TILE)])


def _make_agg():
    mesh = plsc.VectorSubcoreMesh(core_axis_name="c", subcore_axis_name="s")
    return pl.kernel(
        _agg_body,
        out_type=jax.ShapeDtypeStruct((NC, N_ACC, DP), jnp.float32),
        mesh=mesh,
        scratch_types=[
            pltpu.VMEM((NCHUNK, CHUNK), jnp.int32),
            pltpu.VMEM((DCHUNKS, CHUNK), jnp.int32),
            pltpu.VMEM((CHUNK, DP), jnp.float32),
            pltpu.VMEM((CHUNK, DP), jnp.float32),
            pltpu.VMEM((CHUNK, DP), jnp.float32),
            pltpu.VMEM((CHUNK, DP), jnp.float32),
            pltpu.VMEM_SHARED((N_ACC, DP), jnp.float32),
            pltpu.SemaphoreType.DMA,
            pltpu.SemaphoreType.DMA,
            pltpu.SemaphoreType.DMA,
            pltpu.SemaphoreType.DMA,
            pltpu.SemaphoreType.DMA,
            pltpu.SemaphoreType.DMA,
            pltpu.SemaphoreType.DMA,
            pltpu.SemaphoreType.DMA,
        ],
        compiler_params=pltpu.CompilerParams(use_tc_tiling_on_sc=False),
    )


def _layer1_body(acc_ref, feats_ref, ws_ref, wn_ref, b_ref, out_ref):
    s = acc_ref[0] + acc_ref[1]
    deg = jnp.maximum(s[:, D_IN], 1.0)
    hn = s[:, :D_IN] / deg[:, None]
    h = (jnp.dot(feats_ref[...], ws_ref[...], preferred_element_type=jnp.float32)
         + jnp.dot(hn, wn_ref[...], preferred_element_type=jnp.float32)
         + b_ref[...])
    h = jnp.maximum(h, 0.0)
    col = lax.broadcasted_iota(jnp.int32, (h.shape[0], DP - D_IN), 1)
    pad = jnp.where(col == 0, 1.0, 0.0).astype(jnp.float32)
    out_ref[...] = jnp.concatenate([h, pad], axis=1)


def _layer2_body(acc_ref, h_ref, ws_ref, wn_ref, b_ref, out_ref):
    s = acc_ref[0] + acc_ref[1]
    deg = jnp.maximum(s[:, D_IN], 1.0)
    hn = s[:, :D_IN] / deg[:, None]
    out_ref[...] = (jnp.dot(h_ref[:, :D_IN], ws_ref[...],
                            preferred_element_type=jnp.float32)
                    + jnp.dot(hn, wn_ref[...],
                              preferred_element_type=jnp.float32)
                    + b_ref[...])


BLK = 1000


def _make_layer1():
    grid = (N // BLK,)
    return pl.pallas_call(
        _layer1_body,
        grid=grid,
        in_specs=[
            pl.BlockSpec((NC, BLK, DP), lambda i: (0, i, 0)),
            pl.BlockSpec((BLK, D_IN), lambda i: (i, 0)),
            pl.BlockSpec((D_IN, D_HID), lambda i: (0, 0)),
            pl.BlockSpec((D_IN, D_HID), lambda i: (0, 0)),
            pl.BlockSpec((1, D_HID), lambda i: (0, 0)),
        ],
        out_specs=pl.BlockSpec((BLK, DP), lambda i: (i, 0)),
        out_shape=jax.ShapeDtypeStruct((N, DP), jnp.float32),
    )


def _make_layer2():
    grid = (N // BLK,)
    return pl.pallas_call(
        _layer2_body,
        grid=grid,
        in_specs=[
            pl.BlockSpec((NC, BLK, DP), lambda i: (0, i, 0)),
            pl.BlockSpec((BLK, DP), lambda i: (i, 0)),
            pl.BlockSpec((D_HID, D_OUT), lambda i: (0, 0)),
            pl.BlockSpec((D_HID, D_OUT), lambda i: (0, 0)),
            pl.BlockSpec((1, D_OUT), lambda i: (0, 0)),
        ],
        out_specs=pl.BlockSpec((BLK, D_OUT), lambda i: (i, 0)),
        out_shape=jax.ShapeDtypeStruct((N, D_OUT), jnp.float32),
    )


@jax.jit
def kernel(feats, edge_index, Ws1, Wn1, b1, Ws2, Wn2, b2):
    npad = EPW_PAD - EPW
    src = jnp.pad(edge_index[0].astype(jnp.int32).reshape(NW, EPW),
                  ((0, 0), (0, npad))).reshape(NW, NCHUNK, CHUNK)
    dst = jnp.pad(edge_index[1].astype(jnp.int32).reshape(NW, EPW),
                  ((0, 0), (0, npad)),
                  constant_values=N).reshape(NW, NCHUNK, CHUNK)
    pad = jnp.concatenate(
        [jnp.ones((N, 1), jnp.float32), jnp.zeros((N, DP - D_IN - 1), jnp.float32)],
        axis=1)
    feats_p = jnp.concatenate([feats, pad], axis=1)
    zeros = jnp.zeros((ROWS_PER_TILE, DP), jnp.float32)

    agg = _make_agg()
    acc1 = agg(feats_p, src, dst, zeros)
    h1p = _make_layer1()(acc1, feats, Ws1, Wn1, b1.reshape(1, D_HID))
    acc2 = agg(h1p, src, dst, zeros)
    out = _make_layer2()(acc2, h1p, Ws2, Wn2, b2.reshape(1, D_OUT))
    return out
